# Initial kernel scaffold; baseline (speedup 1.0000x reference)
import jax
import jax.numpy as jnp
from jax.experimental import pallas as pl


def kernel(x, edge_index, edge_attr, Wx1, as1, ad1, We1, ae1, b1, Wx2, as2, ad2, We2, ae2, b2, Wx3, as3, ad3, We3, ae3, b3, Wf, bf):
    raise NotImplementedError("write your pallas kernel here")


# speedup vs baseline: 15.0186x; 15.0186x over previous
import functools

import jax
import jax.numpy as jnp
from jax import lax
from jax.experimental import pallas as pl
from jax.experimental.pallas import tpu as pltpu
from jax.experimental.pallas import tpu_sc as plsc

N_NODES = 10000
N_EDGES = 320000
NPAD = 10240
NC = 2
NS = 16
NW = NC * NS
LANES = 16
CHUNK = 80
EPT = N_EDGES // NW
EPS = N_EDGES // NS
HP = 16

_mesh = plsc.VectorSubcoreMesh(core_axis_name="c", subcore_axis_name="s",
                               num_cores=NC, num_subcores=NS)


def _wid():
    return lax.axis_index("s") * NC + lax.axis_index("c")


def _prep_body(nchunks, x_raw0, x_raw1, b, Wx, As, Ad, xout, hout, asout, adout,
               *, relu_in):
    if x_raw1 is None:
        xin = x_raw0[...]
    else:
        xin = jnp.concatenate([x_raw0[...], x_raw1[...]], axis=1)
    if relu_in:
        xin = jnp.maximum(xin + b[...], 0.0)
    if xout is not None:
        xout[...] = xin
    h = jnp.dot(xin, Wx[...], preferred_element_type=jnp.float32)
    F = h.shape[1] // nchunks
    for k in range(nchunks):
        hout[k] = h[:, k * F:(k + 1) * F]
    asout[...] = jnp.dot(h, As[...], preferred_element_type=jnp.float32)
    adout[...] = jnp.dot(h, Ad[...], preferred_element_type=jnp.float32)


def _make_prep(din, dout, nchunks, relu_in, has_x1, W, rows=256):
    grid = (NPAD // rows,)
    F = dout // nchunks
    full = lambda shp: pl.BlockSpec(shp, lambda i: (0,) * len(shp))
    row = lambda w: pl.BlockSpec((rows, w), lambda i: (i, 0))
    in_specs = [pl.BlockSpec((rows, din // (2 if has_x1 else 1)), lambda i: (i, 0))]
    if has_x1:
        in_specs.append(pl.BlockSpec((rows, din // 2), lambda i: (i, 0)))
    in_specs += [full((1, din)), full((din, dout)), full((dout, W)),
                 full((dout, W))]
    out_specs = []
    out_shapes = []
    if relu_in:
        out_specs.append(row(din))
        out_shapes.append(jax.ShapeDtypeStruct((NPAD, din), jnp.float32))
    out_specs.append(pl.BlockSpec((nchunks, rows, F), lambda i: (0, i, 0)))
    out_shapes.append(jax.ShapeDtypeStruct((nchunks, NPAD, F), jnp.float32))
    out_specs += [row(W), row(W)]
    out_shapes += [jax.ShapeDtypeStruct((NPAD, W), jnp.float32)] * 2

    def body(*refs):
        if relu_in:
            if has_x1:
                x0, x1, b, Wx, As, Ad, xout, hout, aso, ado = refs
            else:
                x0, b, Wx, As, Ad, xout, hout, aso, ado = refs
                x1 = None
            _prep_body(nchunks, x0, x1, b, Wx, As, Ad, xout, hout, aso, ado,
                       relu_in=True)
        else:
            x0, b, Wx, As, Ad, hout, aso, ado = refs
            _prep_body(nchunks, x0, None, b, Wx, As, Ad, None, hout, aso, ado,
                       relu_in=False)

    return pl.pallas_call(
        body, grid=grid, in_specs=in_specs, out_specs=tuple(out_specs),
        out_shape=tuple(out_shapes))


def _edge_logits_kernel(eattr_ref, ae_ref, o1_ref, o2_ref, o3_ref):
    r = jnp.dot(eattr_ref[...], ae_ref[...],
                preferred_element_type=jnp.float32)
    o1_ref[...] = r[:, 0:16]
    o2_ref[...] = r[:, 16:32]
    o3_ref[...] = r[:, 32:64]


def _edge_logits(edge_attr, ae_all):
    rows = 2000
    roww = lambda w: pl.BlockSpec((rows, w), lambda i: (i, 0))
    return pl.pallas_call(
        _edge_logits_kernel,
        grid=(N_EDGES // rows,),
        in_specs=[pl.BlockSpec((rows, edge_attr.shape[1]), lambda i: (i, 0)),
                  pl.BlockSpec(ae_all.shape, lambda i: (0, 0))],
        out_specs=(roww(16), roww(16), roww(32)),
        out_shape=(jax.ShapeDtypeStruct((N_EDGES, 16), jnp.float32),
                   jax.ShapeDtypeStruct((N_EDGES, 16), jnp.float32),
                   jax.ShapeDtypeStruct((N_EDGES, 32), jnp.float32)),
    )(edge_attr, ae_all)


def _final_kernel(*refs):
    (x1_ref, x2_ref), parts, (b3_ref, w1, w2), ws, (bf_ref, out_ref) = (
        refs[0:2], refs[2:10], refs[10:13], refs[13:21], refs[21:23])
    acc = jnp.dot(x1_ref[...], w1[...], preferred_element_type=jnp.float32)
    acc += jnp.dot(x2_ref[...], w2[...], preferred_element_type=jnp.float32)
    for k in range(8):
        x3k = jnp.maximum(parts[k][...] + b3_ref[:, k * 96:(k + 1) * 96], 0.0)
        acc += jnp.dot(x3k, ws[k][...], preferred_element_type=jnp.float32)
    out_ref[...] = jax.nn.sigmoid(acc + bf_ref[...])


def _final(x1, x2, acc3, b3, Wf, bf):
    rows = 256
    full = lambda a: pl.BlockSpec(a.shape, lambda i: (0, 0))
    row = lambda w: pl.BlockSpec((rows, w), lambda i: (i, 0))
    w1 = Wf[0:256]
    w2 = Wf[256:512]
    wcs = [Wf[512 + k * 96: 512 + (k + 1) * 96] for k in range(8)]
    b3r = b3.reshape(1, 768)
    bfr = jnp.broadcast_to(bf.reshape(1, 1), (1, 8))
    w1p = jnp.pad(w1, ((0, 0), (0, 7)))
    w2p = jnp.pad(w2, ((0, 0), (0, 7)))
    wcsp = [jnp.pad(w, ((0, 0), (0, 7))) for w in wcs]
    return pl.pallas_call(
        _final_kernel,
        grid=(NPAD // rows,),
        in_specs=[row(256), row(256)] + [row(96)] * 8 +
                 [full(b3r), full(w1p), full(w2p)] + [full(w) for w in wcsp] +
                 [full(bfr)],
        out_specs=row(8),
        out_shape=jax.ShapeDtypeStruct((NPAD, 8), jnp.float32),
    )(x1, x2, *acc3, b3r, w1p, w2p, *wcsp, bfr)


def _zero_spmem(accum, zbuf, nrows_per_tile, ncols):
    sub = lax.axis_index("s")
    zch = zbuf.shape[0]

    def zrow(r, _):
        for v in range(ncols // LANES):
            zbuf[r, pl.ds(v * LANES, LANES)] = jnp.zeros((LANES,), jnp.float32)
        return 0

    lax.fori_loop(0, zch, zrow, 0)
    for k in range(nrows_per_tile // zch):
        pltpu.sync_copy(zbuf, accum.at[pl.ds(sub * nrows_per_tile + k * zch, zch)])


def _pass_a(src, dst, a_e, a_s, a_d, W):
    @functools.partial(
        pl.kernel, mesh=_mesh,
        compiler_params=pltpu.CompilerParams(use_tc_tiling_on_sc=False),
        out_type=(jax.ShapeDtypeStruct((N_EDGES, W), jnp.float32),
                  jax.ShapeDtypeStruct((NC, NPAD, W), jnp.float32)),
        scratch_types=[
            pltpu.VMEM((CHUNK,), jnp.int32),
            pltpu.VMEM((CHUNK,), jnp.int32),
            pltpu.VMEM((CHUNK, W), jnp.float32),
            pltpu.VMEM((CHUNK, W), jnp.float32),
            pltpu.VMEM((CHUNK, W), jnp.float32),
            pltpu.VMEM((CHUNK, W), jnp.float32),
            pltpu.VMEM((CHUNK, W), jnp.float32),
            pltpu.VMEM_SHARED((NPAD, W), jnp.float32),
            pltpu.SemaphoreType.DMA,
            pltpu.SemaphoreType.DMA,
        ])
    def k(src_h, dst_h, ae_h, as_h, ad_h, ex_h, den_h,
          srcv, dstv, aev, asr, adr, exb, zbuf, accum, sem1, sem2):
        w = _wid()
        core = lax.axis_index("c")
        sub = lax.axis_index("s")
        _zero_spmem(accum, zbuf, NPAD // NS, W)
        plsc.subcore_barrier()

        def chunk(i, _):
            b = w * EPT + i * CHUNK
            pltpu.sync_copy(src_h.at[pl.ds(b, CHUNK)], srcv)
            pltpu.sync_copy(dst_h.at[pl.ds(b, CHUNK)], dstv)
            pltpu.sync_copy(ae_h.at[pl.ds(b, CHUNK), :], aev)
            ca = pltpu.async_copy(as_h.at[srcv], asr, sem1)
            cb = pltpu.async_copy(ad_h.at[dstv], adr, sem2)
            ca.wait()
            cb.wait()

            def row(r, _):
                for u in range(W // LANES):
                    s = pl.ds(u * LANES, LANES)
                    alpha = asr[r, s] + adr[r, s] + aev[r, s]
                    alpha = jnp.where(alpha > 0, alpha, 0.2 * alpha)
                    exb[r, s] = jnp.exp(alpha)
                return 0

            lax.fori_loop(0, CHUNK, row, 0)
            pltpu.sync_copy(exb, ex_h.at[pl.ds(b, CHUNK), :])
            pltpu.sync_copy(exb, accum.at[dstv], add=True)
            return 0

        lax.fori_loop(0, EPT // CHUNK, chunk, 0)
        plsc.subcore_barrier()
        rpt = NPAD // NS
        pltpu.sync_copy(accum.at[pl.ds(sub * rpt, rpt)],
                        den_h.at[core, pl.ds(sub * rpt, rpt), :])

    return k(src, dst, a_e, a_s, a_d)


def _pass_b(ex, dst, den, W):
    @functools.partial(
        pl.kernel, mesh=_mesh,
        compiler_params=pltpu.CompilerParams(use_tc_tiling_on_sc=False),
        out_type=jax.ShapeDtypeStruct((N_EDGES, W), jnp.float32),
        scratch_types=[
            pltpu.VMEM((CHUNK,), jnp.int32),
            pltpu.VMEM((CHUNK, W), jnp.float32),
            pltpu.VMEM((CHUNK, W), jnp.float32),
            pltpu.VMEM((CHUNK, W), jnp.float32),
            pltpu.SemaphoreType.DMA,
            pltpu.SemaphoreType.DMA,
        ])
    def k(ex_h, dst_h, d0_h, d1_h, co_h, dstv, exb, d0, d1, sem1, sem2):
        w = _wid()

        def chunk(i, _):
            b = w * EPT + i * CHUNK
            pltpu.sync_copy(dst_h.at[pl.ds(b, CHUNK)], dstv)
            pltpu.sync_copy(ex_h.at[pl.ds(b, CHUNK), :], exb)
            ca = pltpu.async_copy(d0_h.at[dstv], d0, sem1)
            cb = pltpu.async_copy(d1_h.at[dstv], d1, sem2)
            ca.wait()
            cb.wait()

            def row(r, _):
                for u in range(W // LANES):
                    s = pl.ds(u * LANES, LANES)
                    exb[r, s] = exb[r, s] / (d0[r, s] + d1[r, s] + 1e-16)
                return 0

            lax.fori_loop(0, CHUNK, row, 0)
            pltpu.sync_copy(exb, co_h.at[pl.ds(b, CHUNK), :])
            return 0

        lax.fori_loop(0, EPT // CHUNK, chunk, 0)

    return k(ex, dst, den[0], den[1])


def _pass_c(src, dst, coeff, h2, F, group_base, group_mul, ktab):
    nv = F // LANES

    @functools.partial(
        pl.kernel, mesh=_mesh,
        compiler_params=pltpu.CompilerParams(use_tc_tiling_on_sc=False),
        out_type=(jax.ShapeDtypeStruct((NPAD, F), jnp.float32),
                  jax.ShapeDtypeStruct((NPAD, F), jnp.float32)),
        scratch_types=[
            pltpu.VMEM((CHUNK,), jnp.int32),
            pltpu.VMEM((CHUNK,), jnp.int32),
            pltpu.VMEM((CHUNK, LANES), jnp.float32),
            pltpu.VMEM((CHUNK, F), jnp.float32),
            pltpu.VMEM((CHUNK, F), jnp.float32),
            pltpu.VMEM((CHUNK, F), jnp.float32),
            pltpu.VMEM_SHARED((NPAD, F), jnp.float32),
            pltpu.SemaphoreType.DMA,
        ])
    def k(src_h, dst_h, co_h, h_h, out0_h, out1_h,
          srcv, dstv, cob, hb, msg, zbuf, accum, sem):
        core = lax.axis_index("c")
        sub = lax.axis_index("s")
        _zero_spmem(accum, zbuf, NPAD // NS, F)
        plsc.subcore_barrier()
        goff = pl.multiple_of((group_base + group_mul * core) * 8, 8)

        def chunk(i, _):
            b = sub * EPS + i * CHUNK
            pltpu.sync_copy(src_h.at[pl.ds(b, CHUNK)], srcv)
            pltpu.sync_copy(dst_h.at[pl.ds(b, CHUNK)], dstv)
            pltpu.sync_copy(co_h.at[pl.ds(b, CHUNK), pl.ds(goff, 8)],
                            cob.at[:, pl.ds(0, 8)])
            for v in range(CHUNK // LANES):
                srcv[pl.ds(v * LANES, LANES)] = (
                    srcv[pl.ds(v * LANES, LANES)] + core * NPAD)
            pltpu.async_copy(h_h.at[srcv], hb, sem).wait()

            def make_rowloop(ks):
                def row(r, _):
                    crow = cob[r, :]
                    for v in range(nv):
                        cv = jnp.broadcast_to(crow[ks[v]], (LANES,))
                        msg[r, pl.ds(v * LANES, LANES)] = (
                            hb[r, pl.ds(v * LANES, LANES)] * cv)
                    return 0
                return row

            if ktab(0) == ktab(1):
                lax.fori_loop(0, CHUNK, make_rowloop(ktab(0)), 0)
            else:
                @pl.when(core == 0)
                def _():
                    lax.fori_loop(0, CHUNK, make_rowloop(ktab(0)), 0)

                @pl.when(core == 1)
                def _():
                    lax.fori_loop(0, CHUNK, make_rowloop(ktab(1)), 0)

            pltpu.sync_copy(msg, accum.at[dstv], add=True)
            return 0

        lax.fori_loop(0, EPS // CHUNK, chunk, 0)
        plsc.subcore_barrier()
        rpt = NPAD // NS
        sl = pl.ds(sub * rpt, rpt)

        @pl.when(core == 0)
        def _():
            pltpu.sync_copy(accum.at[sl], out0_h.at[sl])

        @pl.when(core == 1)
        def _():
            pltpu.sync_copy(accum.at[sl], out1_h.at[sl])

    return k(src, dst, coeff, h2)


def _att_mat(att, H, C, gs):
    W = 8 * (H // gs)
    m = (jnp.eye(H, dtype=jnp.float32)[:, None, :] * att[:, :, None]
         ).reshape(H * C, H)
    out = jnp.zeros((H * C, W), jnp.float32)
    for hd in range(H):
        out = out.at[:, 8 * (hd // gs) + hd % gs].set(m[:, hd])
    return out


def _ae_mat(We, ae, H, C, gs):
    W = 8 * (H // gs)
    m = (We.reshape(22, H, C) * ae[None, :, :]).sum(-1)
    out = jnp.zeros((22, W), jnp.float32)
    for hd in range(H):
        out = out.at[:, 8 * (hd // gs) + hd % gs].set(m[:, hd])
    return out


def kernel(x, edge_index, edge_attr, Wx1, as1, ad1, We1, ae1, b1,
           Wx2, as2, ad2, We2, ae2, b2, Wx3, as3, ad3, We3, ae3, b3, Wf, bf):
    src = edge_index[0]
    dst = edge_index[1]
    xp = jnp.pad(x, ((0, NPAD - N_NODES), (0, 0)))

    ae_all = jnp.concatenate(
        [_ae_mat(We1, ae1, 8, 32, 4), _ae_mat(We2, ae2, 8, 32, 4),
         _ae_mat(We3, ae3, 12, 64, 3)], axis=1)
    ae_1, ae_2, ae_3 = _edge_logits(edge_attr, ae_all)

    prep1 = _make_prep(128, 256, 2, relu_in=False, has_x1=False, W=16)
    h1c, as_1, ad_1 = prep1(xp, jnp.zeros((1, 128), jnp.float32), Wx1,
                            _att_mat(as1, 8, 32, 4), _att_mat(ad1, 8, 32, 4))
    ex1, den1 = _pass_a(src, dst, ae_1, as_1, ad_1, 16)
    co1 = _pass_b(ex1, dst, den1, 16)
    kt8 = lambda c: [v // 2 for v in range(8)]
    o1a, o1b = _pass_c(src, dst, co1, h1c.reshape(2 * NPAD, 128), 128,
                       0, 1, kt8)

    prep2 = _make_prep(256, 256, 2, relu_in=True, has_x1=True, W=16)
    x1, h2c, as_2, ad_2 = prep2(o1a, o1b, b1.reshape(1, 256), Wx2,
                                _att_mat(as2, 8, 32, 4), _att_mat(ad2, 8, 32, 4))
    ex2, den2 = _pass_a(src, dst, ae_2, as_2, ad_2, 16)
    co2 = _pass_b(ex2, dst, den2, 16)
    o2a, o2b = _pass_c(src, dst, co2, h2c.reshape(2 * NPAD, 128), 128,
                       0, 1, kt8)

    prep3 = _make_prep(256, 768, 8, relu_in=True, has_x1=True, W=32)
    x2, h3c, as_3, ad_3 = prep3(o2a, o2b, b2.reshape(1, 256), Wx3,
                                _att_mat(as3, 12, 64, 3), _att_mat(ad3, 12, 64, 3))
    ex3, den3 = _pass_a(src, dst, ae_3, as_3, ad_3, 32)
    co3 = _pass_b(ex3, dst, den3, 32)
    kt96 = lambda c: [(6 * c + v) // 4 for v in range(6)]
    o3 = []
    for t in range(4):
        hj = lax.slice_in_dim(h3c, 2 * t, 2 * t + 2).reshape(2 * NPAD, 96)
        oja, ojb = _pass_c(src, dst, co3, hj, 96, t, 0, kt96)
        o3 += [oja, ojb]

    out = _final(x1, x2, o3, b3, Wf, bf)
    return out[:N_NODES, :1]


# speedup vs baseline: 24.0779x; 1.6032x over previous
import functools

import jax
import jax.numpy as jnp
from jax import lax
from jax.experimental import pallas as pl
from jax.experimental.pallas import tpu as pltpu
from jax.experimental.pallas import tpu_sc as plsc

N_NODES = 10000
N_EDGES = 320000
NPAD = 10240
NC = 2
NS = 16
NW = NC * NS
LANES = 16
CHUNK = 80
EPT = N_EDGES // NW
EPS = N_EDGES // NS
HP = 16

_mesh = plsc.VectorSubcoreMesh(core_axis_name="c", subcore_axis_name="s",
                               num_cores=NC, num_subcores=NS)


def _wid():
    return lax.axis_index("s") * NC + lax.axis_index("c")


def _prep_body(nchunks, x_raw0, x_raw1, b, Wx, As, Ad, xout, hout, asout, adout,
               *, relu_in):
    if x_raw1 is None:
        xin = x_raw0[...]
    else:
        xin = jnp.concatenate([x_raw0[...], x_raw1[...]], axis=1)
    if relu_in:
        xin = jnp.maximum(xin + b[...], 0.0)
    if xout is not None:
        xout[...] = xin
    h = jnp.dot(xin, Wx[...], preferred_element_type=jnp.float32)
    F = h.shape[1] // nchunks
    for k in range(nchunks):
        hout[k] = h[:, k * F:(k + 1) * F]
    asout[...] = jnp.dot(h, As[...], preferred_element_type=jnp.float32)
    adout[...] = jnp.dot(h, Ad[...], preferred_element_type=jnp.float32)


def _make_prep(din, dout, nchunks, relu_in, has_x1, W, rows=256):
    grid = (NPAD // rows,)
    F = dout // nchunks
    full = lambda shp: pl.BlockSpec(shp, lambda i: (0,) * len(shp))
    row = lambda w: pl.BlockSpec((rows, w), lambda i: (i, 0))
    in_specs = [pl.BlockSpec((rows, din // (2 if has_x1 else 1)), lambda i: (i, 0))]
    if has_x1:
        in_specs.append(pl.BlockSpec((rows, din // 2), lambda i: (i, 0)))
    in_specs += [full((1, din)), full((din, dout)), full((dout, W)),
                 full((dout, W))]
    out_specs = []
    out_shapes = []
    if relu_in:
        out_specs.append(row(din))
        out_shapes.append(jax.ShapeDtypeStruct((NPAD, din), jnp.float32))
    out_specs.append(pl.BlockSpec((nchunks, rows, F), lambda i: (0, i, 0)))
    out_shapes.append(jax.ShapeDtypeStruct((nchunks, NPAD, F), jnp.float32))
    out_specs += [row(W), row(W)]
    out_shapes += [jax.ShapeDtypeStruct((NPAD, W), jnp.float32)] * 2

    def body(*refs):
        if relu_in:
            if has_x1:
                x0, x1, b, Wx, As, Ad, xout, hout, aso, ado = refs
            else:
                x0, b, Wx, As, Ad, xout, hout, aso, ado = refs
                x1 = None
            _prep_body(nchunks, x0, x1, b, Wx, As, Ad, xout, hout, aso, ado,
                       relu_in=True)
        else:
            x0, b, Wx, As, Ad, hout, aso, ado = refs
            _prep_body(nchunks, x0, None, b, Wx, As, Ad, None, hout, aso, ado,
                       relu_in=False)

    return pl.pallas_call(
        body, grid=grid, in_specs=in_specs, out_specs=tuple(out_specs),
        out_shape=tuple(out_shapes))


def _edge_logits_kernel(eattr_ref, ae_ref, o1_ref, o2_ref, o3_ref):
    r = jnp.dot(eattr_ref[...], ae_ref[...],
                preferred_element_type=jnp.float32)
    o1_ref[...] = r[:, 0:16]
    o2_ref[...] = r[:, 16:32]
    o3_ref[...] = r[:, 32:64]


def _edge_logits(edge_attr, ae_all):
    rows = 2000
    roww = lambda w: pl.BlockSpec((rows, w), lambda i: (i, 0))
    return pl.pallas_call(
        _edge_logits_kernel,
        grid=(N_EDGES // rows,),
        in_specs=[pl.BlockSpec((rows, edge_attr.shape[1]), lambda i: (i, 0)),
                  pl.BlockSpec(ae_all.shape, lambda i: (0, 0))],
        out_specs=(roww(16), roww(16), roww(32)),
        out_shape=(jax.ShapeDtypeStruct((N_EDGES, 16), jnp.float32),
                   jax.ShapeDtypeStruct((N_EDGES, 16), jnp.float32),
                   jax.ShapeDtypeStruct((N_EDGES, 32), jnp.float32)),
    )(edge_attr, ae_all)


def _final_kernel(*refs):
    (x1_ref, x2_ref), parts, (b3_ref, w1, w2), ws, (bf_ref, out_ref) = (
        refs[0:2], refs[2:10], refs[10:13], refs[13:21], refs[21:23])
    acc = jnp.dot(x1_ref[...], w1[...], preferred_element_type=jnp.float32)
    acc += jnp.dot(x2_ref[...], w2[...], preferred_element_type=jnp.float32)
    for k in range(8):
        x3k = jnp.maximum(parts[k][...] + b3_ref[:, k * 96:(k + 1) * 96], 0.0)
        acc += jnp.dot(x3k, ws[k][...], preferred_element_type=jnp.float32)
    out_ref[...] = jax.nn.sigmoid(acc + bf_ref[...])


def _final(x1, x2, acc3, b3, Wf, bf):
    rows = 256
    full = lambda a: pl.BlockSpec(a.shape, lambda i: (0, 0))
    row = lambda w: pl.BlockSpec((rows, w), lambda i: (i, 0))
    w1 = Wf[0:256]
    w2 = Wf[256:512]
    wcs = [Wf[512 + k * 96: 512 + (k + 1) * 96] for k in range(8)]
    b3r = b3.reshape(1, 768)
    bfr = jnp.broadcast_to(bf.reshape(1, 1), (1, 8))
    w1p = jnp.pad(w1, ((0, 0), (0, 7)))
    w2p = jnp.pad(w2, ((0, 0), (0, 7)))
    wcsp = [jnp.pad(w, ((0, 0), (0, 7))) for w in wcs]
    return pl.pallas_call(
        _final_kernel,
        grid=(NPAD // rows,),
        in_specs=[row(256), row(256)] + [row(96)] * 8 +
                 [full(b3r), full(w1p), full(w2p)] + [full(w) for w in wcsp] +
                 [full(bfr)],
        out_specs=row(8),
        out_shape=jax.ShapeDtypeStruct((NPAD, 8), jnp.float32),
    )(x1, x2, *acc3, b3r, w1p, w2p, *wcsp, bfr)


def _zero_spmem(accum, zbuf, nrows_per_tile, ncols):
    sub = lax.axis_index("s")
    zch = zbuf.shape[0]

    def zrow(r, _):
        for v in range(ncols // LANES):
            zbuf[r, pl.ds(v * LANES, LANES)] = jnp.zeros((LANES,), jnp.float32)
        return 0

    lax.fori_loop(0, zch, zrow, 0)
    for k in range(nrows_per_tile // zch):
        pltpu.sync_copy(zbuf, accum.at[pl.ds(sub * nrows_per_tile + k * zch, zch)])


def _pass_a(src, dst, a_e, a_s, a_d, W):
    @functools.partial(
        pl.kernel, mesh=_mesh,
        compiler_params=pltpu.CompilerParams(use_tc_tiling_on_sc=False),
        out_type=(jax.ShapeDtypeStruct((N_EDGES, W), jnp.float32),
                  jax.ShapeDtypeStruct((NC, NPAD, W), jnp.float32)),
        scratch_types=[
            pltpu.VMEM((CHUNK,), jnp.int32),
            pltpu.VMEM((CHUNK,), jnp.int32),
            pltpu.VMEM((CHUNK, W), jnp.float32),
            pltpu.VMEM((CHUNK, W), jnp.float32),
            pltpu.VMEM((CHUNK, W), jnp.float32),
            pltpu.VMEM((CHUNK, W), jnp.float32),
            pltpu.VMEM((CHUNK, W), jnp.float32),
            pltpu.VMEM_SHARED((NPAD, W), jnp.float32),
            pltpu.SemaphoreType.DMA,
            pltpu.SemaphoreType.DMA,
        ])
    def k(src_h, dst_h, ae_h, as_h, ad_h, ex_h, den_h,
          srcv, dstv, aev, asr, adr, exb, zbuf, accum, sem1, sem2):
        w = _wid()
        core = lax.axis_index("c")
        sub = lax.axis_index("s")
        _zero_spmem(accum, zbuf, NPAD // NS, W)
        plsc.subcore_barrier()

        def chunk(i, _):
            b = w * EPT + i * CHUNK
            pltpu.sync_copy(src_h.at[pl.ds(b, CHUNK)], srcv)
            pltpu.sync_copy(dst_h.at[pl.ds(b, CHUNK)], dstv)
            pltpu.sync_copy(ae_h.at[pl.ds(b, CHUNK), :], aev)
            ca = pltpu.async_copy(as_h.at[srcv], asr, sem1)
            cb = pltpu.async_copy(ad_h.at[dstv], adr, sem2)
            ca.wait()
            cb.wait()

            def row(r, _):
                for u in range(W // LANES):
                    s = pl.ds(u * LANES, LANES)
                    alpha = asr[r, s] + adr[r, s] + aev[r, s]
                    alpha = jnp.where(alpha > 0, alpha, 0.2 * alpha)
                    exb[r, s] = jnp.exp(alpha)
                return 0

            lax.fori_loop(0, CHUNK, row, 0)
            pltpu.sync_copy(exb, ex_h.at[pl.ds(b, CHUNK), :])
            pltpu.sync_copy(exb, accum.at[dstv], add=True)
            return 0

        lax.fori_loop(0, EPT // CHUNK, chunk, 0)
        plsc.subcore_barrier()
        rpt = NPAD // NS
        pltpu.sync_copy(accum.at[pl.ds(sub * rpt, rpt)],
                        den_h.at[core, pl.ds(sub * rpt, rpt), :])

    return k(src, dst, a_e, a_s, a_d)


def _pass_b(ex, dst, den, W):
    @functools.partial(
        pl.kernel, mesh=_mesh,
        compiler_params=pltpu.CompilerParams(use_tc_tiling_on_sc=False),
        out_type=jax.ShapeDtypeStruct((N_EDGES, W), jnp.float32),
        scratch_types=[
            pltpu.VMEM((CHUNK,), jnp.int32),
            pltpu.VMEM((CHUNK, W), jnp.float32),
            pltpu.VMEM((CHUNK, W), jnp.float32),
            pltpu.VMEM((CHUNK, W), jnp.float32),
            pltpu.SemaphoreType.DMA,
            pltpu.SemaphoreType.DMA,
        ])
    def k(ex_h, dst_h, d0_h, d1_h, co_h, dstv, exb, d0, d1, sem1, sem2):
        w = _wid()

        def chunk(i, _):
            b = w * EPT + i * CHUNK
            pltpu.sync_copy(dst_h.at[pl.ds(b, CHUNK)], dstv)
            pltpu.sync_copy(ex_h.at[pl.ds(b, CHUNK), :], exb)
            ca = pltpu.async_copy(d0_h.at[dstv], d0, sem1)
            cb = pltpu.async_copy(d1_h.at[dstv], d1, sem2)
            ca.wait()
            cb.wait()

            def row(r, _):
                for u in range(W // LANES):
                    s = pl.ds(u * LANES, LANES)
                    exb[r, s] = exb[r, s] / (d0[r, s] + d1[r, s] + 1e-16)
                return 0

            lax.fori_loop(0, CHUNK, row, 0)
            pltpu.sync_copy(exb, co_h.at[pl.ds(b, CHUNK), :])
            return 0

        lax.fori_loop(0, EPT // CHUNK, chunk, 0)

    return k(ex, dst, den[0], den[1])


def _pass_c(src, dst, coeff, h2, F, group_base, group_mul, ktab):
    nv = F // LANES
    nchunks = EPS // CHUNK
    npairs = nchunks // 2
    assert npairs * 2 == nchunks

    @functools.partial(
        pl.kernel, mesh=_mesh,
        compiler_params=pltpu.CompilerParams(use_tc_tiling_on_sc=False),
        out_type=(jax.ShapeDtypeStruct((NPAD, F), jnp.float32),
                  jax.ShapeDtypeStruct((NPAD, F), jnp.float32)),
        scratch_types=(
            [pltpu.VMEM((CHUNK,), jnp.int32)] * 6 +
            [pltpu.VMEM((CHUNK, LANES), jnp.float32)] * 2 +
            [pltpu.VMEM((CHUNK, F), jnp.float32)] * 4 +
            [pltpu.VMEM_SHARED((NPAD, F), jnp.float32)] +
            [pltpu.SemaphoreType.DMA] * 6
        ))
    def k(src_h, dst_h, co_h, h_h, out0_h, out1_h,
          srcv0, srcv1, dstv0, dstv1, dsc0, dsc1, cob0, cob1, hb0, hb1,
          msg0, msg1, accum, isem0, isem1, gsem0, gsem1, ssem0, ssem1):
        core = lax.axis_index("c")
        sub = lax.axis_index("s")
        srcv = (srcv0, srcv1)
        dstv = (dstv0, dstv1)
        dsc = (dsc0, dsc1)
        cob = (cob0, cob1)
        hb = (hb0, hb1)
        msg = (msg0, msg1)
        isem = (isem0, isem1)
        gsem = (gsem0, gsem1)
        ssem = (ssem0, ssem1)
        _zero_spmem(accum, msg0, NPAD // NS, F)
        plsc.subcore_barrier()
        goff = pl.multiple_of((group_base + group_mul * core) * 8, 8)
        base = sub * EPS

        def issue_loads(p, b):
            pltpu.async_copy(src_h.at[pl.ds(b, CHUNK)], srcv[p], isem[p])
            pltpu.async_copy(dst_h.at[pl.ds(b, CHUNK)], dstv[p], isem[p])
            pltpu.async_copy(co_h.at[pl.ds(b, CHUNK), pl.ds(goff, 8)],
                             cob[p].at[:, pl.ds(0, 8)], isem[p])

        def wait_loads(p, b):
            pltpu.make_async_copy(src_h.at[pl.ds(b, CHUNK)], srcv[p],
                                  isem[p]).wait()
            pltpu.make_async_copy(dst_h.at[pl.ds(b, CHUNK)], dstv[p],
                                  isem[p]).wait()
            pltpu.make_async_copy(co_h.at[pl.ds(b, CHUNK), pl.ds(goff, 8)],
                                  cob[p].at[:, pl.ds(0, 8)], isem[p]).wait()

        def fix_and_gather(p):
            for v in range(CHUNK // LANES):
                srcv[p][pl.ds(v * LANES, LANES)] = (
                    srcv[p][pl.ds(v * LANES, LANES)] + core * NPAD)
            pltpu.async_copy(h_h.at[srcv[p]], hb[p], gsem[p])

        def wait_gather(p):
            pltpu.make_async_copy(h_h.at[srcv[p]], hb[p], gsem[p]).wait()

        def issue_scatter(p):
            for v in range(CHUNK // LANES):
                dsc[p][pl.ds(v * LANES, LANES)] = dstv[p][pl.ds(v * LANES, LANES)]
            pltpu.async_copy(msg[p], accum.at[dsc[p]], ssem[p], add=True)

        def wait_scatter(p):
            pltpu.make_async_copy(msg[p], accum.at[dsc[p]], ssem[p]).wait()

        def make_rowloop(p, ks):
            def row(r, _):
                crow = cob[p][r, :]
                for v in range(nv):
                    cv = jnp.broadcast_to(crow[ks[v]], (LANES,))
                    msg[p][r, pl.ds(v * LANES, LANES)] = (
                        hb[p][r, pl.ds(v * LANES, LANES)] * cv)
                return 0
            return row

        def compute(p):
            if ktab(0) == ktab(1):
                lax.fori_loop(0, CHUNK, make_rowloop(p, ktab(0)), 0)
            else:
                @pl.when(core == 0)
                def _():
                    lax.fori_loop(0, CHUNK, make_rowloop(p, ktab(0)), 0)

                @pl.when(core == 1)
                def _():
                    lax.fori_loop(0, CHUNK, make_rowloop(p, ktab(1)), 0)

        for p in (0, 1):
            issue_loads(p, base + p * CHUNK)
        for p in (0, 1):
            wait_loads(p, base + p * CHUNK)
            fix_and_gather(p)

        def pair(io, _):
            for p in (0, 1):
                i = 2 * io + p
                wait_gather(p)

                @pl.when(io > 0)
                def _():
                    wait_scatter(p)

                compute(p)
                issue_scatter(p)

                @pl.when(io < npairs - 1)
                def _():
                    issue_loads(p, base + (i + 2) * CHUNK)
                    wait_loads(p, base + (i + 2) * CHUNK)
                    fix_and_gather(p)
            return 0

        lax.fori_loop(0, npairs, pair, 0)
        wait_scatter(0)
        wait_scatter(1)
        plsc.subcore_barrier()
        rpt = NPAD // NS
        sl = pl.ds(sub * rpt, rpt)

        @pl.when(core == 0)
        def _():
            pltpu.sync_copy(accum.at[sl], out0_h.at[sl])

        @pl.when(core == 1)
        def _():
            pltpu.sync_copy(accum.at[sl], out1_h.at[sl])

    return k(src, dst, coeff, h2)


def _att_mat(att, H, C, gs):
    W = 8 * (H // gs)
    m = (jnp.eye(H, dtype=jnp.float32)[:, None, :] * att[:, :, None]
         ).reshape(H * C, H)
    out = jnp.zeros((H * C, W), jnp.float32)
    for hd in range(H):
        out = out.at[:, 8 * (hd // gs) + hd % gs].set(m[:, hd])
    return out


def _ae_mat(We, ae, H, C, gs):
    W = 8 * (H // gs)
    m = (We.reshape(22, H, C) * ae[None, :, :]).sum(-1)
    out = jnp.zeros((22, W), jnp.float32)
    for hd in range(H):
        out = out.at[:, 8 * (hd // gs) + hd % gs].set(m[:, hd])
    return out


def kernel(x, edge_index, edge_attr, Wx1, as1, ad1, We1, ae1, b1,
           Wx2, as2, ad2, We2, ae2, b2, Wx3, as3, ad3, We3, ae3, b3, Wf, bf):
    src = edge_index[0]
    dst = edge_index[1]
    xp = jnp.pad(x, ((0, NPAD - N_NODES), (0, 0)))

    ae_all = jnp.concatenate(
        [_ae_mat(We1, ae1, 8, 32, 4), _ae_mat(We2, ae2, 8, 32, 4),
         _ae_mat(We3, ae3, 12, 64, 3)], axis=1)
    ae_1, ae_2, ae_3 = _edge_logits(edge_attr, ae_all)

    prep1 = _make_prep(128, 256, 2, relu_in=False, has_x1=False, W=16)
    h1c, as_1, ad_1 = prep1(xp, jnp.zeros((1, 128), jnp.float32), Wx1,
                            _att_mat(as1, 8, 32, 4), _att_mat(ad1, 8, 32, 4))
    ex1, den1 = _pass_a(src, dst, ae_1, as_1, ad_1, 16)
    co1 = _pass_b(ex1, dst, den1, 16)
    kt8 = lambda c: [v // 2 for v in range(8)]
    o1a, o1b = _pass_c(src, dst, co1, h1c.reshape(2 * NPAD, 128), 128,
                       0, 1, kt8)

    prep2 = _make_prep(256, 256, 2, relu_in=True, has_x1=True, W=16)
    x1, h2c, as_2, ad_2 = prep2(o1a, o1b, b1.reshape(1, 256), Wx2,
                                _att_mat(as2, 8, 32, 4), _att_mat(ad2, 8, 32, 4))
    ex2, den2 = _pass_a(src, dst, ae_2, as_2, ad_2, 16)
    co2 = _pass_b(ex2, dst, den2, 16)
    o2a, o2b = _pass_c(src, dst, co2, h2c.reshape(2 * NPAD, 128), 128,
                       0, 1, kt8)

    prep3 = _make_prep(256, 768, 8, relu_in=True, has_x1=True, W=32)
    x2, h3c, as_3, ad_3 = prep3(o2a, o2b, b2.reshape(1, 256), Wx3,
                                _att_mat(as3, 12, 64, 3), _att_mat(ad3, 12, 64, 3))
    ex3, den3 = _pass_a(src, dst, ae_3, as_3, ad_3, 32)
    co3 = _pass_b(ex3, dst, den3, 32)
    kt96 = lambda c: [(6 * c + v) // 4 for v in range(6)]
    o3 = []
    for t in range(4):
        hj = lax.slice_in_dim(h3c, 2 * t, 2 * t + 2).reshape(2 * NPAD, 96)
        oja, ojb = _pass_c(src, dst, co3, hj, 96, t, 0, kt96)
        o3 += [oja, ojb]

    out = _final(x1, x2, o3, b3, Wf, bf)
    return out[:N_NODES, :1]
